# baseline (speedup 1.0000x reference)
import functools

import jax
import jax.numpy as jnp
from jax import lax
from jax.experimental import pallas as pl
from jax.experimental.pallas import tpu as pltpu
from jax.experimental.pallas import tpu_sc as plsc

NUM_CLASSES = 1000000
FEATURE_DIM = 64
BATCH = 16384

NC = 2
NS = 16
L = 16
NW = NC * NS
B_PER_W = BATCH // NW
IDX_CHUNK = 128
N_CHUNKS = B_PER_W // IDX_CHUNK
VECS_PER_ROW = FEATURE_DIM // L


def _center_loss_body(feat_hbm, labels_hbm, centers_hbm, out_hbm,
                      idx_v, rows_v, feat_v, acc_v, gsem, fsem):
    wid = lax.axis_index("s") * NC + lax.axis_index("c")
    base = wid * B_PER_W

    pltpu.sync_copy(labels_hbm.at[pl.ds(wid * N_CHUNKS, N_CHUNKS)], idx_v)

    fcopy = pltpu.async_copy(feat_hbm.at[pl.ds(base, B_PER_W)], feat_v, fsem)
    gathers = [
        pltpu.async_copy(
            centers_hbm.at[idx_v.at[j]],
            rows_v.at[pl.ds(j * IDX_CHUNK, IDX_CHUNK)],
            gsem,
        )
        for j in range(N_CHUNKS)
    ]
    fcopy.wait()
    for g in gathers:
        g.wait()

    def row_step(r, acc):
        for j in range(VECS_PER_ROW):
            f = feat_v[r, pl.ds(j * L, L)]
            c = rows_v[r, pl.ds(j * L, L)]
            d = f - c
            acc = acc + d * d
        return acc

    acc = lax.fori_loop(0, B_PER_W, row_step, jnp.zeros((L,), jnp.float32))
    acc_v[...] = acc
    pltpu.sync_copy(acc_v, out_hbm.at[wid])


@jax.jit
def _center_loss(features, labels2d, centers):
    mesh = plsc.VectorSubcoreMesh(
        core_axis_name="c", subcore_axis_name="s",
        num_cores=NC, num_subcores=NS,
    )
    partials = pl.kernel(
        _center_loss_body,
        out_type=jax.ShapeDtypeStruct((NW, L), jnp.float32),
        mesh=mesh,
        scratch_types=[
            pltpu.VMEM((N_CHUNKS, IDX_CHUNK), jnp.int32),
            pltpu.VMEM((B_PER_W, FEATURE_DIM), jnp.float32),
            pltpu.VMEM((B_PER_W, FEATURE_DIM), jnp.float32),
            pltpu.VMEM((L,), jnp.float32),
            pltpu.SemaphoreType.DMA,
            pltpu.SemaphoreType.DMA,
        ],
        compiler_params=pltpu.CompilerParams(use_tc_tiling_on_sc=False),
    )(features, labels2d, centers)
    return jnp.sum(partials) * (1.0 / BATCH)


def kernel(features, labels, centers):
    labels2d = labels.astype(jnp.int32).reshape(NW * N_CHUNKS, IDX_CHUNK)
    return _center_loss(features, labels2d, centers)


# speedup vs baseline: 1.6204x; 1.6204x over previous
import jax
import jax.numpy as jnp
from jax import lax
from jax.experimental import pallas as pl
from jax.experimental.pallas import tpu as pltpu
from jax.experimental.pallas import tpu_sc as plsc

NUM_CLASSES = 1000000
FEATURE_DIM = 64
BATCH = 16384

NC = 2
NS = 16
L = 16
NW = NC * NS
B_PER_W = BATCH // NW
CHUNK = 64
N_CHUNKS = B_PER_W // CHUNK
VECS_PER_ROW = FEATURE_DIM // L


def _center_loss_body(feat_hbm, labels_hbm, centers_hbm, out_hbm,
                      lab_v, feat_v, g_v, acc_v, fsem, gsem0, gsem1):
    wid = lax.axis_index("s") * NC + lax.axis_index("c")
    base = wid * B_PER_W

    pltpu.sync_copy(labels_hbm.at[pl.ds(base, B_PER_W)], lab_v)
    fcopy = pltpu.async_copy(feat_hbm.at[pl.ds(base, B_PER_W)], feat_v, fsem)

    gsems = [gsem0, gsem1]

    def fire(ch):
        b = ch % 2
        descs = []
        for g in range(CHUNK // L):
            l16 = lab_v[pl.ds(ch * CHUNK + g * L, L)]
            for q in range(L):
                descs.append(pltpu.async_copy(
                    centers_hbm.at[l16[q]], g_v.at[b, g * L + q], gsems[b]))
        return descs

    def compute_chunk(ch, acc):
        b = ch % 2

        def grp_step(ii, acc):
            for q in range(L):
                i = ii * L + q
                gi = ch * CHUNK + i
                for j in range(VECS_PER_ROW):
                    f = feat_v[gi, pl.ds(j * L, L)]
                    c = g_v[b, i, pl.ds(j * L, L)]
                    d = f - c
                    acc = acc + d * d
            return acc

        return lax.fori_loop(0, CHUNK // L, grp_step, acc)

    pending = fire(0)
    fcopy.wait()
    acc = jnp.zeros((L,), jnp.float32)
    for ch in range(N_CHUNKS):
        for d in pending:
            d.wait()
        if ch + 1 < N_CHUNKS:
            pending = fire(ch + 1)
        acc = compute_chunk(ch, acc)

    acc_v[...] = acc
    pltpu.sync_copy(acc_v, out_hbm.at[wid])


@jax.jit
def _center_loss(features, labels32, centers):
    mesh = plsc.VectorSubcoreMesh(
        core_axis_name="c", subcore_axis_name="s",
        num_cores=NC, num_subcores=NS,
    )
    partials = pl.kernel(
        _center_loss_body,
        out_type=jax.ShapeDtypeStruct((NW, L), jnp.float32),
        mesh=mesh,
        scratch_types=[
            pltpu.VMEM((B_PER_W,), jnp.int32),
            pltpu.VMEM((B_PER_W, FEATURE_DIM), jnp.float32),
            pltpu.VMEM((2, CHUNK, FEATURE_DIM), jnp.float32),
            pltpu.VMEM((L,), jnp.float32),
            pltpu.SemaphoreType.DMA,
            pltpu.SemaphoreType.DMA,
            pltpu.SemaphoreType.DMA,
        ],
    )(features, labels32, centers)
    return jnp.sum(partials) * (1.0 / BATCH)


def kernel(features, labels, centers):
    labels32 = labels.astype(jnp.int32)
    return _center_loss(features, labels32, centers)


# speedup vs baseline: 1.6653x; 1.0277x over previous
import jax
import jax.numpy as jnp
from jax import lax
from jax.experimental import pallas as pl
from jax.experimental.pallas import tpu as pltpu
from jax.experimental.pallas import tpu_sc as plsc

NUM_CLASSES = 1000000
FEATURE_DIM = 64
BATCH = 16384

NC = 2
NS = 16
L = 16
NW = NC * NS
B_PER_W = BATCH // NW
CHUNK = 128
N_CHUNKS = B_PER_W // CHUNK
GRPS_PER_CHUNK = CHUNK // L
VECS_PER_ROW = FEATURE_DIM // L


def _center_loss_body(feat_hbm, labels_hbm, centers_hbm, out_hbm,
                      lab_v, feat_v, rows_v, acc_v, fsem, gsem0, gsem1):
    wid = lax.axis_index("s") * NC + lax.axis_index("c")
    base = wid * B_PER_W

    pltpu.sync_copy(labels_hbm.at[pl.ds(base, B_PER_W)], lab_v)
    fcopy = pltpu.async_copy(feat_hbm.at[pl.ds(base, B_PER_W)], feat_v, fsem)

    gsems = [gsem0, gsem1]

    def fire(ch):
        b = ch % 2

        def fire_group(g, carry):
            l16 = lab_v[pl.ds(ch * CHUNK + g * L, L)]
            for q in range(L):
                pltpu.async_copy(
                    centers_hbm.at[l16[q]], rows_v.at[b].at[g * L + q],
                    gsems[b])
            return carry

        lax.fori_loop(0, GRPS_PER_CHUNK, fire_group, 0)

    def drain(ch):
        b = ch % 2
        pltpu.make_async_copy(
            centers_hbm.at[pl.ds(0, CHUNK)], rows_v.at[b], gsems[b]).wait()

    def compute_chunk(ch, acc):
        b = ch % 2

        def grp_step(ii, acc):
            for q in range(L):
                for j in range(VECS_PER_ROW):
                    f = feat_v[ch * CHUNK + ii * L + q, pl.ds(j * L, L)]
                    c = rows_v[b, ii * L + q, pl.ds(j * L, L)]
                    d = f - c
                    acc = acc + d * d
            return acc

        return lax.fori_loop(0, GRPS_PER_CHUNK, grp_step, acc)

    fire(0)
    fcopy.wait()
    acc = jnp.zeros((L,), jnp.float32)
    for ch in range(N_CHUNKS):
        drain(ch)
        if ch + 1 < N_CHUNKS:
            fire(ch + 1)
        acc = compute_chunk(ch, acc)

    acc_v[...] = acc
    pltpu.sync_copy(acc_v, out_hbm.at[wid])


@jax.jit
def _center_loss(features, labels32, centers):
    mesh = plsc.VectorSubcoreMesh(
        core_axis_name="c", subcore_axis_name="s",
        num_cores=NC, num_subcores=NS,
    )
    partials = pl.kernel(
        _center_loss_body,
        out_type=jax.ShapeDtypeStruct((NW, L), jnp.float32),
        mesh=mesh,
        scratch_types=[
            pltpu.VMEM((B_PER_W,), jnp.int32),
            pltpu.VMEM((B_PER_W, FEATURE_DIM), jnp.float32),
            pltpu.VMEM((2, CHUNK, FEATURE_DIM), jnp.float32),
            pltpu.VMEM((L,), jnp.float32),
            pltpu.SemaphoreType.DMA,
            pltpu.SemaphoreType.DMA,
            pltpu.SemaphoreType.DMA,
        ],
    )(features, labels32, centers)
    return jnp.sum(partials) * (1.0 / BATCH)


def kernel(features, labels, centers):
    labels32 = labels.astype(jnp.int32)
    return _center_loss(features, labels32, centers)


# speedup vs baseline: 1.6744x; 1.0055x over previous
import jax
import jax.numpy as jnp
from jax import lax
from jax.experimental import pallas as pl
from jax.experimental.pallas import tpu as pltpu
from jax.experimental.pallas import tpu_sc as plsc

NUM_CLASSES = 1000000
FEATURE_DIM = 64
BATCH = 16384

NC = 2
NS = 16
L = 16
NW = NC * NS
B_PER_W = BATCH // NW
CHUNK = 128
N_CHUNKS = B_PER_W // CHUNK
GRPS_PER_CHUNK = CHUNK // L
VECS_PER_ROW = FEATURE_DIM // L


def _center_loss_body(feat_hbm, labels_hbm, centers_hbm, out_hbm,
                      lab_v, feat_v, rows_v, acc_v, fsem, gsem0, gsem1):
    wid = lax.axis_index("s") * NC + lax.axis_index("c")
    base = wid * B_PER_W

    pltpu.sync_copy(labels_hbm.at[pl.ds(base, B_PER_W)], lab_v)
    fcopy = pltpu.async_copy(feat_hbm.at[pl.ds(base, B_PER_W)], feat_v, fsem)

    gsems = [gsem0, gsem1]

    def fire(ch):
        b = ch % 2

        def fire_group(g, carry):
            l16 = lab_v[pl.ds(ch * CHUNK + g * L, L)]
            for q in range(L):
                pltpu.async_copy(
                    centers_hbm.at[l16[q]], rows_v.at[b].at[g * L + q],
                    gsems[b])
            return carry

        lax.fori_loop(0, GRPS_PER_CHUNK, fire_group, 0)

    def drain(ch):
        b = ch % 2
        pltpu.make_async_copy(
            centers_hbm.at[pl.ds(0, CHUNK)], rows_v.at[b], gsems[b]).wait()

    def compute_chunk(ch, acc):
        b = ch % 2

        def grp_step(ii, acc):
            for q in range(L):
                for j in range(VECS_PER_ROW):
                    f = feat_v[ch * CHUNK + ii * L + q, pl.ds(j * L, L)]
                    c = rows_v[b, ii * L + q, pl.ds(j * L, L)]
                    d = f - c
                    acc = acc + d * d
            return acc

        return lax.fori_loop(0, GRPS_PER_CHUNK, grp_step, acc)

    fire(0)
    fcopy.wait()
    acc = jnp.zeros((L,), jnp.float32)
    for ch in range(N_CHUNKS):
        drain(ch)
        if ch + 1 < N_CHUNKS:
            fire(ch + 1)
        acc = compute_chunk(ch, acc)

    acc_v[...] = acc
    pltpu.sync_copy(acc_v, out_hbm.at[wid])


@jax.jit
def _center_loss(features, labels32, centers):
    mesh = plsc.VectorSubcoreMesh(
        core_axis_name="c", subcore_axis_name="s",
        num_cores=NC, num_subcores=NS,
    )
    partials = pl.kernel(
        _center_loss_body,
        out_type=jax.ShapeDtypeStruct((NW, L), jnp.float32),
        mesh=mesh,
        scratch_types=[
            pltpu.VMEM((B_PER_W,), jnp.int32),
            pltpu.VMEM((B_PER_W, FEATURE_DIM), jnp.float32),
            pltpu.VMEM((2, CHUNK, FEATURE_DIM), jnp.float32),
            pltpu.VMEM((L,), jnp.float32),
            pltpu.SemaphoreType.DMA,
            pltpu.SemaphoreType.DMA,
            pltpu.SemaphoreType.DMA,
        ],
        compiler_params=pltpu.CompilerParams(skip_device_barrier=True),
    )(features, labels32, centers)
    return jnp.sum(partials) * (1.0 / BATCH)


def kernel(features, labels, centers):
    labels32 = labels.astype(jnp.int32)
    return _center_loss(features, labels32, centers)
